# baseline (speedup 1.0000x reference)
import functools

import jax
import jax.numpy as jnp
from jax import lax
from jax.experimental import pallas as pl
from jax.experimental.pallas import tpu as pltpu
from jax.experimental.pallas import tpu_sc as plsc

_B = 4
_S = 2048
_D = 1024
_N = _B * _S
_NC = 2
_NS = 16
_NW = _NC * _NS
_PER_W = _N // _NW
_CHUNK = 32
_STEPS = _PER_W // _CHUNK
_LANES = 16


def _embed_kernel(ids_hbm, table_hbm, pos_hbm, out_hbm,
                  idx_v, pos_v, rows_v, sem):
    wid = lax.axis_index("s") * _NC + lax.axis_index("c")
    base = wid * _PER_W
    s_base = base % _S
    for step in range(_STEPS):
        off = base + step * _CHUNK
        s_off = s_base + step * _CHUNK
        pltpu.sync_copy(ids_hbm.at[pl.ds(off, _CHUNK)], idx_v)
        pltpu.sync_copy(pos_hbm.at[pl.ds(s_off, _CHUNK)], pos_v)
        pltpu.async_copy(table_hbm.at[idx_v], rows_v, sem).wait()

        def _add_row(i, _):
            for c in range(_D // _LANES):
                sl = pl.ds(c * _LANES, _LANES)
                rows_v[i, sl] = rows_v[i, sl] + pos_v[i, sl]
            return _

        lax.fori_loop(0, _CHUNK, _add_row, None)
        pltpu.sync_copy(rows_v, out_hbm.at[pl.ds(off, _CHUNK)])


def kernel(input_ids, token_table, pos_table):
    ids_flat = input_ids.reshape(_N).astype(jnp.int32)
    mesh = plsc.VectorSubcoreMesh(core_axis_name="c", subcore_axis_name="s")
    run = functools.partial(
        pl.kernel,
        out_type=jax.ShapeDtypeStruct((_N, _D), jnp.float32),
        mesh=mesh,
        scratch_types=[
            pltpu.VMEM((_CHUNK,), jnp.int32),
            pltpu.VMEM((_CHUNK, _D), jnp.float32),
            pltpu.VMEM((_CHUNK, _D), jnp.float32),
            pltpu.SemaphoreType.DMA,
        ],
    )(_embed_kernel)
    out = run(ids_flat, token_table, pos_table)
    return out.reshape(_B, _S, _D)


# speedup vs baseline: 1.2946x; 1.2946x over previous
import functools

import jax
import jax.numpy as jnp
from jax import lax
from jax.experimental import pallas as pl
from jax.experimental.pallas import tpu as pltpu
from jax.experimental.pallas import tpu_sc as plsc

_B = 4
_S = 2048
_D = 1024
_N = _B * _S
_NC = 2
_NS = 16
_NW = _NC * _NS
_PER_W = _N // _NW
_CHUNK = 16
_STEPS = _PER_W // _CHUNK
_LANES = 16


def _embed_kernel(ids_hbm, table_hbm, pos_hbm, out_hbm,
                  idx_v, pos_v, rows_v, semg, semp, sems):
    wid = lax.axis_index("s") * _NC + lax.axis_index("c")
    base = wid * _PER_W
    s_base = base % _S
    pltpu.sync_copy(ids_hbm.at[pl.ds(base, _PER_W)], idx_v)

    gather_d = {}
    pos_d = {}
    store_d = {}

    def issue(g):
        p = g % 2
        if g >= 2:
            store_d[g - 2].wait()
        idxs = idx_v[pl.ds(g * _CHUNK, _CHUNK)]
        gather_d[g] = pltpu.async_copy(table_hbm.at[idxs], rows_v.at[p], semg.at[p])
        pos_d[g] = pltpu.async_copy(
            pos_hbm.at[pl.ds(s_base + g * _CHUNK, _CHUNK)], pos_v.at[p], semp.at[p])

    def process(g):
        p = g % 2
        gather_d[g].wait()
        pos_d[g].wait()

        def _add_row(i, carry):
            for c in range(_D // _LANES):
                sl = pl.ds(c * _LANES, _LANES)
                plsc.addupdate(rows_v.at[p, i, sl], pos_v[p, i, sl])
            return carry

        lax.fori_loop(0, _CHUNK, _add_row, None)
        store_d[g] = pltpu.async_copy(
            rows_v.at[p], out_hbm.at[pl.ds(base + g * _CHUNK, _CHUNK)], sems.at[p])

    issue(0)
    for g in range(1, _STEPS):
        issue(g)
        process(g - 1)
    process(_STEPS - 1)
    store_d[_STEPS - 2].wait()
    store_d[_STEPS - 1].wait()


def kernel(input_ids, token_table, pos_table):
    ids_flat = input_ids.reshape(_N).astype(jnp.int32)
    mesh = plsc.VectorSubcoreMesh(core_axis_name="c", subcore_axis_name="s")
    run = functools.partial(
        pl.kernel,
        out_type=jax.ShapeDtypeStruct((_N, _D), jnp.float32),
        mesh=mesh,
        scratch_types=[
            pltpu.VMEM((_PER_W,), jnp.int32),
            pltpu.VMEM((2, _CHUNK, _D), jnp.float32),
            pltpu.VMEM((2, _CHUNK, _D), jnp.float32),
            pltpu.SemaphoreType.DMA((2,)),
            pltpu.SemaphoreType.DMA((2,)),
            pltpu.SemaphoreType.DMA((2,)),
        ],
    )(_embed_kernel)
    out = run(ids_flat, token_table, pos_table)
    return out.reshape(_B, _S, _D)


# speedup vs baseline: 1.2951x; 1.0004x over previous
import functools

import jax
import jax.numpy as jnp
from jax import lax
from jax.experimental import pallas as pl
from jax.experimental.pallas import tpu as pltpu
from jax.experimental.pallas import tpu_sc as plsc

_B = 4
_S = 2048
_D = 1024
_N = _B * _S
_NC = 2
_NS = 16
_NW = _NC * _NS
_PER_W = _N // _NW
_CHUNK = 16
_STEPS = _PER_W // _CHUNK
_LANES = 16


def _embed_kernel(ids_hbm, table_hbm, pos_hbm, out_hbm,
                  idx_v, pos_v, rows_v, semg, semp, sems):
    wid = lax.axis_index("s") * _NC + lax.axis_index("c")
    base = wid * _PER_W
    b_idx = base // _S
    s_base = base % _S
    pltpu.sync_copy(ids_hbm.at[b_idx, pl.ds(s_base, _PER_W)], idx_v)

    gather_d = {}
    pos_d = {}
    store_d = {}

    def issue(g):
        p = g % 2
        if g >= 2:
            store_d[g - 2].wait()
        idxs = idx_v[pl.ds(g * _CHUNK, _CHUNK)]
        gather_d[g] = pltpu.async_copy(table_hbm.at[idxs], rows_v.at[p], semg.at[p])
        pos_d[g] = pltpu.async_copy(
            pos_hbm.at[pl.ds(s_base + g * _CHUNK, _CHUNK)], pos_v.at[p], semp.at[p])

    def process(g):
        p = g % 2
        gather_d[g].wait()
        pos_d[g].wait()

        def _add_row(i, carry):
            for c in range(_D // _LANES):
                sl = pl.ds(c * _LANES, _LANES)
                plsc.addupdate(rows_v.at[p, i, sl], pos_v[p, i, sl])
            return carry

        lax.fori_loop(0, _CHUNK, _add_row, None)
        store_d[g] = pltpu.async_copy(
            rows_v.at[p], out_hbm.at[b_idx, pl.ds(s_base + g * _CHUNK, _CHUNK)],
            sems.at[p])

    issue(0)
    for g in range(1, _STEPS):
        issue(g)
        process(g - 1)
    process(_STEPS - 1)
    store_d[_STEPS - 2].wait()
    store_d[_STEPS - 1].wait()


def kernel(input_ids, token_table, pos_table):
    mesh = plsc.VectorSubcoreMesh(core_axis_name="c", subcore_axis_name="s")
    run = functools.partial(
        pl.kernel,
        out_type=jax.ShapeDtypeStruct((_B, _S, _D), jnp.float32),
        mesh=mesh,
        scratch_types=[
            pltpu.VMEM((_PER_W,), jnp.int32),
            pltpu.VMEM((2, _CHUNK, _D), jnp.float32),
            pltpu.VMEM((2, _CHUNK, _D), jnp.float32),
            pltpu.SemaphoreType.DMA((2,)),
            pltpu.SemaphoreType.DMA((2,)),
            pltpu.SemaphoreType.DMA((2,)),
        ],
    )(_embed_kernel)
    return run(input_ids.astype(jnp.int32), token_table, pos_table)


# speedup vs baseline: 1.6338x; 1.2616x over previous
import functools

import jax
import jax.numpy as jnp
from jax import lax
from jax.experimental import pallas as pl
from jax.experimental.pallas import tpu as pltpu
from jax.experimental.pallas import tpu_sc as plsc

_B = 4
_S = 2048
_D = 1024
_N = _B * _S
_NC = 2
_NS = 16
_NW = _NC * _NS
_PER_W = _N // _NW
_CHUNK = 16
_STEPS = _PER_W // _CHUNK
_NBUF = 3
_LANES = 16


def _embed_kernel(ids_hbm, table_hbm, pos_hbm, out_hbm,
                  idx_v, pos_v, rows_v, semg, semp, sems):
    wid = lax.axis_index("s") * _NC + lax.axis_index("c")
    base = wid * _PER_W
    b_idx = base // _S
    s_base = base % _S
    pltpu.sync_copy(ids_hbm.at[b_idx, pl.ds(s_base, _PER_W)], idx_v)

    def issue(g):
        p = lax.rem(g, _NBUF)
        idxs = idx_v[pl.ds(g * _CHUNK, _CHUNK)]
        pltpu.async_copy(table_hbm.at[idxs], rows_v.at[p], semg.at[p])
        pltpu.async_copy(
            pos_hbm.at[pl.ds(s_base + g * _CHUNK, _CHUNK)], pos_v.at[p], semp.at[p])

    def wait_store(j):
        p = lax.rem(j, _NBUF)
        pltpu.make_async_copy(
            rows_v.at[p], out_hbm.at[0, pl.ds(0, _CHUNK)], sems.at[p]).wait()

    def process(j):
        p = lax.rem(j, _NBUF)
        pltpu.make_async_copy(
            table_hbm.at[pl.ds(0, _CHUNK)], rows_v.at[p], semg.at[p]).wait()
        pltpu.make_async_copy(
            pos_hbm.at[pl.ds(0, _CHUNK)], pos_v.at[p], semp.at[p]).wait()

        def _add_row(i, carry):
            for c in range(_D // _LANES):
                sl = pl.ds(c * _LANES, _LANES)
                plsc.addupdate(rows_v.at[p, i, sl], pos_v[p, i, sl])
            return carry

        lax.fori_loop(0, _CHUNK, _add_row, None)
        pltpu.async_copy(
            rows_v.at[p], out_hbm.at[b_idx, pl.ds(s_base + j * _CHUNK, _CHUNK)],
            sems.at[p])

    def body(g, carry):
        @pl.when(g >= _NBUF)
        def _():
            wait_store(g - _NBUF)

        issue(g)
        process(g - 1)
        return carry

    issue(0)
    lax.fori_loop(1, _STEPS, body, None)
    process(_STEPS - 1)
    for j in range(_STEPS - _NBUF, _STEPS):
        wait_store(j)


def kernel(input_ids, token_table, pos_table):
    mesh = plsc.VectorSubcoreMesh(core_axis_name="c", subcore_axis_name="s")
    run = functools.partial(
        pl.kernel,
        out_type=jax.ShapeDtypeStruct((_B, _S, _D), jnp.float32),
        mesh=mesh,
        scratch_types=[
            pltpu.VMEM((_PER_W,), jnp.int32),
            pltpu.VMEM((_NBUF, _CHUNK, _D), jnp.float32),
            pltpu.VMEM((_NBUF, _CHUNK, _D), jnp.float32),
            pltpu.SemaphoreType.DMA((_NBUF,)),
            pltpu.SemaphoreType.DMA((_NBUF,)),
            pltpu.SemaphoreType.DMA((_NBUF,)),
        ],
    )(_embed_kernel)
    return run(input_ids.astype(jnp.int32), token_table, pos_table)
